# baseline (speedup 1.0000x reference)
import functools

import jax
import jax.numpy as jnp
from jax import lax
from jax.experimental import pallas as pl
from jax.experimental.pallas import tpu as pltpu

_F32 = jnp.float32


def _dotg(a, b, ca, cb):
    return lax.dot_general(a, b, (((ca,), (cb,)), ((), ())),
                           preferred_element_type=_F32)


def _tabgnn_kernel(a_ref, x_ref, w0_ref, b0t_ref, w1_ref, w0g_ref, w0x_ref,
                   bt_ref, mw0_ref, mb0_ref, wl_ref, bl_ref, o_ref,
                   scr_a, scr_hw, acc_ref, *, t1_steps, r1, r2,
                   num_graph_types):
    g = pl.program_id(0)
    t = pl.program_id(1)

    @pl.when(t < t1_steps)
    def _layer1():
        a_blk = a_ref[0]
        scr_a[pl.ds(t * r1, r1), :] = a_blk
        tt = _dotg(x_ref[...], a_blk, 0, 1)
        h1t = jnp.maximum(_dotg(w0_ref[0], tt, 0, 0) + b0t_ref[0], 0.0)
        scr_hw[pl.ds(t * r1, r1), :] = _dotg(h1t, w1_ref[0], 0, 0)

    @pl.when(t >= t1_steps)
    def _layer2():
        rows = pl.ds((t - t1_steps) * r2, r2)
        a_rows = scr_a[rows, :]
        embt = _dotg(scr_hw[...], a_rows, 0, 1)
        contrib = _dotg(embt, w0g_ref[0], 0, 0)

        @pl.when(g == 0)
        def _():
            acc_ref[rows, :] = (
                jnp.dot(x_ref[rows, :], w0x_ref[...],
                        preferred_element_type=_F32)
                + bt_ref[...] + contrib)

        @pl.when(g > 0)
        def _():
            acc_ref[rows, :] += contrib

        @pl.when(g == num_graph_types - 1)
        def _():
            hm = jnp.maximum(acc_ref[rows, :], 0.0)
            hm = jnp.maximum(
                jnp.dot(hm, mw0_ref[...], preferred_element_type=_F32)
                + mb0_ref[...], 0.0)
            o_ref[...] = (jnp.sum(hm * wl_ref[...], axis=1, keepdims=True)
                          + bl_ref[...])


def kernel(a_hats, x, gnn_w_0, gnn_w_1, gnn_b_0, gnn_b_1, w0x, w0g, b0,
           mlp_w_0, mlp_b_0, mlp_w_1, mlp_b_1):
    G, N, _ = a_hats.shape
    Fin = x.shape[1]
    H = gnn_w_0.shape[2]
    Fout = gnn_w_1.shape[2]
    H0 = w0x.shape[1]
    Hm = mlp_w_0.shape[1]

    r1 = 256 if N % 256 == 0 else N
    t1_steps = N // r1
    r2 = 256 if N % 256 == 0 else N
    t2_steps = N // r2

    bias_tot = b0
    for g in range(G):
        bias_tot = bias_tot + jnp.dot(gnn_b_1[g], w0g[g],
                                      preferred_element_type=_F32)
    b0t = jnp.transpose(gnn_b_0, (0, 2, 1))

    flops = int(G * (2 * N * N * Fin + 2 * N * Fin * H + 2 * N * H * Fout
                     + 2 * N * N * Fout + 2 * N * Fout * H0)
                + 2 * N * Fin * H0 + 2 * N * H0 * Hm + 2 * N * Hm)
    bytes_accessed = int(4 * (G * N * N + N * Fin + N + Fin * H0
                              + G * (Fin * H + H + H * Fout + Fout * H0)))

    out = pl.pallas_call(
        functools.partial(_tabgnn_kernel, t1_steps=t1_steps, r1=r1, r2=r2,
                          num_graph_types=G),
        out_shape=jax.ShapeDtypeStruct((N, 1), _F32),
        grid=(G, t1_steps + t2_steps),
        in_specs=[
            pl.BlockSpec((1, r1, N),
                         lambda g, t: (g, jnp.minimum(t, t1_steps - 1), 0)),
            pl.BlockSpec((N, Fin), lambda g, t: (0, 0)),
            pl.BlockSpec((1, Fin, H), lambda g, t: (g, 0, 0)),
            pl.BlockSpec((1, H, 1), lambda g, t: (g, 0, 0)),
            pl.BlockSpec((1, H, Fout), lambda g, t: (g, 0, 0)),
            pl.BlockSpec((1, Fout, H0), lambda g, t: (g, 0, 0)),
            pl.BlockSpec((Fin, H0), lambda g, t: (0, 0)),
            pl.BlockSpec((1, H0), lambda g, t: (0, 0)),
            pl.BlockSpec((H0, Hm), lambda g, t: (0, 0)),
            pl.BlockSpec((1, Hm), lambda g, t: (0, 0)),
            pl.BlockSpec((1, Hm), lambda g, t: (0, 0)),
            pl.BlockSpec((1, 1), lambda g, t: (0, 0)),
        ],
        out_specs=pl.BlockSpec(
            (r2, 1), lambda g, t: (jnp.maximum(t - t1_steps, 0), 0)),
        scratch_shapes=[pltpu.VMEM((N, N), _F32),
                        pltpu.VMEM((N, Fout), _F32),
                        pltpu.VMEM((N, H0), _F32)],
        compiler_params=pltpu.CompilerParams(
            dimension_semantics=("arbitrary", "arbitrary"),
            vmem_limit_bytes=58 * 2**20),
        cost_estimate=pl.CostEstimate(flops=flops, transcendentals=0,
                                      bytes_accessed=bytes_accessed),
    )(a_hats, x, gnn_w_0, b0t, gnn_w_1, w0g, w0x, bias_tot,
      mlp_w_0, mlp_b_0, mlp_w_1, mlp_b_1)
    return out


# speedup vs baseline: 1.4233x; 1.4233x over previous
import functools

import jax
import jax.numpy as jnp
from jax import lax
from jax.experimental import pallas as pl
from jax.experimental.pallas import tpu as pltpu

_F32 = jnp.float32


def _dotg(a, b, ca, cb):
    return lax.dot_general(a, b, (((ca,), (cb,)), ((), ())),
                           preferred_element_type=_F32)


def _tabgnn_kernel(a_ref, x_ref, w0_ref, b0t_ref, w1_ref, w0g_ref, w0x_ref,
                   bt_ref, mw0_ref, mb0_ref, wl_ref, bl_ref, o_ref,
                   scr_a, scr_hw, acc_ref, *, half, num_graph_types):
    p = pl.program_id(0)
    t = pl.program_id(1)
    rows = pl.ds(t * half, half)

    @pl.when(p < num_graph_types)
    def _layer1():
        buf = p % 2
        a_blk = a_ref[0]
        scr_a[buf, rows, :] = a_blk
        tt = _dotg(x_ref[...], a_blk, 0, 1)
        h1t = jnp.maximum(_dotg(w0_ref[0], tt, 0, 0) + b0t_ref[0], 0.0)
        scr_hw[buf, rows, :] = _dotg(h1t, w1_ref[0], 0, 0)

    @pl.when(p > 0)
    def _layer2():
        buf = (p - 1) % 2
        a_rows = scr_a[buf, rows, :]
        embt = _dotg(scr_hw[buf], a_rows, 0, 1)
        contrib = _dotg(embt, w0g_ref[0], 0, 0)

        @pl.when(p == 1)
        def _():
            acc_ref[rows, :] = (
                jnp.dot(x_ref[rows, :], w0x_ref[...],
                        preferred_element_type=_F32)
                + bt_ref[...] + contrib)

        @pl.when(p > 1)
        def _():
            acc_ref[rows, :] += contrib

        @pl.when(p == num_graph_types)
        def _():
            hm = jnp.maximum(acc_ref[rows, :], 0.0)
            hm = jnp.maximum(
                jnp.dot(hm, mw0_ref[...], preferred_element_type=_F32)
                + mb0_ref[...], 0.0)
            o_ref[...] = (jnp.sum(hm * wl_ref[...], axis=1, keepdims=True)
                          + bl_ref[...])


def kernel(a_hats, x, gnn_w_0, gnn_w_1, gnn_b_0, gnn_b_1, w0x, w0g, b0,
           mlp_w_0, mlp_b_0, mlp_w_1, mlp_b_1):
    G, N, _ = a_hats.shape
    Fin = x.shape[1]
    H = gnn_w_0.shape[2]
    Fout = gnn_w_1.shape[2]
    H0 = w0x.shape[1]
    Hm = mlp_w_0.shape[1]

    half = N // 2

    bias_tot = b0
    for g in range(G):
        bias_tot = bias_tot + jnp.dot(gnn_b_1[g], w0g[g],
                                      preferred_element_type=_F32)
    b0t = jnp.transpose(gnn_b_0, (0, 2, 1))

    gm1 = G - 1

    flops = int(G * (2 * N * N * Fin + 2 * N * Fin * H + 2 * N * H * Fout
                     + 2 * N * N * Fout + 2 * N * Fout * H0)
                + 2 * N * Fin * H0 + 2 * N * H0 * Hm + 2 * N * Hm)
    bytes_accessed = int(4 * (G * N * N + N * Fin + N + Fin * H0
                              + G * (Fin * H + H + H * Fout + Fout * H0)))

    out = pl.pallas_call(
        functools.partial(_tabgnn_kernel, half=half, num_graph_types=G),
        out_shape=jax.ShapeDtypeStruct((N, 1), _F32),
        grid=(G + 1, 2),
        in_specs=[
            pl.BlockSpec((1, half, N),
                         lambda p, t: (jnp.minimum(p, gm1),
                                       jnp.where(p > gm1, 1, t), 0)),
            pl.BlockSpec((N, Fin), lambda p, t: (0, 0)),
            pl.BlockSpec((1, Fin, H),
                         lambda p, t: (jnp.minimum(p, gm1), 0, 0)),
            pl.BlockSpec((1, H, 1),
                         lambda p, t: (jnp.minimum(p, gm1), 0, 0)),
            pl.BlockSpec((1, H, Fout),
                         lambda p, t: (jnp.minimum(p, gm1), 0, 0)),
            pl.BlockSpec((1, Fout, H0),
                         lambda p, t: (jnp.maximum(p, 1) - 1, 0, 0)),
            pl.BlockSpec((Fin, H0), lambda p, t: (0, 0)),
            pl.BlockSpec((1, H0), lambda p, t: (0, 0)),
            pl.BlockSpec((H0, Hm), lambda p, t: (0, 0)),
            pl.BlockSpec((1, Hm), lambda p, t: (0, 0)),
            pl.BlockSpec((1, Hm), lambda p, t: (0, 0)),
            pl.BlockSpec((1, 1), lambda p, t: (0, 0)),
        ],
        out_specs=pl.BlockSpec(
            (half, 1), lambda p, t: (jnp.where(p > gm1, t, 0), 0)),
        scratch_shapes=[pltpu.VMEM((2, N, N), _F32),
                        pltpu.VMEM((2, N, Fout), _F32),
                        pltpu.VMEM((N, H0), _F32)],
        compiler_params=pltpu.CompilerParams(
            dimension_semantics=("arbitrary", "arbitrary"),
            vmem_limit_bytes=58 * 2**20),
        cost_estimate=pl.CostEstimate(flops=flops, transcendentals=0,
                                      bytes_accessed=bytes_accessed),
    )(a_hats, x, gnn_w_0, b0t, gnn_w_1, w0g, w0x, bias_tot,
      mlp_w_0, mlp_b_0, mlp_w_1, mlp_b_1)
    return out


# speedup vs baseline: 1.4827x; 1.0417x over previous
import functools

import jax
import jax.numpy as jnp
from jax import lax
from jax.experimental import pallas as pl
from jax.experimental.pallas import tpu as pltpu

_F32 = jnp.float32


def _dotg(a, b, ca, cb):
    return lax.dot_general(a, b, (((ca,), (cb,)), ((), ())),
                           preferred_element_type=_F32)


def _tabgnn_kernel(*refs, nstripe, stripe, num_graph_types):
    a_refs = refs[:nstripe]
    (x_ref, w0_ref, b0t_ref, w1_ref, w0g_ref, w0x_ref, bt_ref,
     mw0_ref, mb0_ref, wl_ref, bl_ref, o_ref, acc_ref) = refs[nstripe:]
    g = pl.program_id(0)

    t1 = _dotg(x_ref[0:stripe, :], a_refs[0][0], 0, 1)
    for q in range(1, nstripe):
        t1 += _dotg(x_ref[q * stripe:(q + 1) * stripe, :], a_refs[q][0],
                    0, 1)
    h1t = jnp.maximum(_dotg(w0_ref[0], t1, 0, 0) + b0t_ref[0], 0.0)
    hw1t = _dotg(w1_ref[0], h1t, 0, 0)
    embt = _dotg(hw1t[:, 0:stripe], a_refs[0][0], 1, 1)
    for q in range(1, nstripe):
        embt += _dotg(hw1t[:, q * stripe:(q + 1) * stripe], a_refs[q][0],
                      1, 1)
    contrib = _dotg(embt, w0g_ref[0], 0, 0)

    @pl.when(g == 0)
    def _():
        acc_ref[...] = (jnp.dot(x_ref[...], w0x_ref[...],
                                preferred_element_type=_F32)
                        + bt_ref[...] + contrib)

    @pl.when(g > 0)
    def _():
        acc_ref[...] += contrib

    @pl.when(g == num_graph_types - 1)
    def _():
        hm = jnp.maximum(acc_ref[...], 0.0)
        hm = jnp.maximum(
            jnp.dot(hm, mw0_ref[...], preferred_element_type=_F32)
            + mb0_ref[...], 0.0)
        o_ref[...] = (jnp.sum(hm * wl_ref[...], axis=1, keepdims=True)
                      + bl_ref[...])


def kernel(a_hats, x, gnn_w_0, gnn_w_1, gnn_b_0, gnn_b_1, w0x, w0g, b0,
           mlp_w_0, mlp_b_0, mlp_w_1, mlp_b_1):
    G, N, _ = a_hats.shape
    Fin = x.shape[1]
    H = gnn_w_0.shape[2]
    Fout = gnn_w_1.shape[2]
    H0 = w0x.shape[1]
    Hm = mlp_w_0.shape[1]

    stripe = 256 if (N % 256 == 0 and N // 256 >= 2) else N
    nstripe = N // stripe

    bias_tot = b0
    for g in range(G):
        bias_tot = bias_tot + jnp.dot(gnn_b_1[g], w0g[g],
                                      preferred_element_type=_F32)
    b0t = jnp.transpose(gnn_b_0, (0, 2, 1))

    flops = int(G * (2 * N * N * Fin + 2 * N * Fin * H + 2 * N * H * Fout
                     + 2 * N * N * Fout + 2 * N * Fout * H0)
                + 2 * N * Fin * H0 + 2 * N * H0 * Hm + 2 * N * Hm)
    bytes_accessed = int(4 * (G * N * N + N * Fin + N + Fin * H0
                              + G * (Fin * H + H + H * Fout + Fout * H0)))

    a_specs = [
        pl.BlockSpec((1, N, stripe), (lambda q: (lambda g: (g, 0, q)))(q))
        for q in range(nstripe)
    ]
    out = pl.pallas_call(
        functools.partial(_tabgnn_kernel, nstripe=nstripe, stripe=stripe,
                          num_graph_types=G),
        out_shape=jax.ShapeDtypeStruct((N, 1), _F32),
        grid=(G,),
        in_specs=a_specs + [
            pl.BlockSpec((N, Fin), lambda g: (0, 0)),
            pl.BlockSpec((1, Fin, H), lambda g: (g, 0, 0)),
            pl.BlockSpec((1, H, 1), lambda g: (g, 0, 0)),
            pl.BlockSpec((1, H, Fout), lambda g: (g, 0, 0)),
            pl.BlockSpec((1, Fout, H0), lambda g: (g, 0, 0)),
            pl.BlockSpec((Fin, H0), lambda g: (0, 0)),
            pl.BlockSpec((1, H0), lambda g: (0, 0)),
            pl.BlockSpec((H0, Hm), lambda g: (0, 0)),
            pl.BlockSpec((1, Hm), lambda g: (0, 0)),
            pl.BlockSpec((1, Hm), lambda g: (0, 0)),
            pl.BlockSpec((1, 1), lambda g: (0, 0)),
        ],
        out_specs=pl.BlockSpec((N, 1), lambda g: (0, 0)),
        scratch_shapes=[pltpu.VMEM((N, H0), _F32)],
        compiler_params=pltpu.CompilerParams(
            dimension_semantics=("arbitrary",),
            vmem_limit_bytes=58 * 2**20),
        cost_estimate=pl.CostEstimate(flops=flops, transcendentals=0,
                                      bytes_accessed=bytes_accessed),
    )(*([a_hats] * nstripe), x, gnn_w_0, b0t, gnn_w_1, w0g, w0x, bias_tot,
      mlp_w_0, mlp_b_0, mlp_w_1, mlp_b_1)
    return out


# speedup vs baseline: 1.5329x; 1.0339x over previous
import functools

import jax
import jax.numpy as jnp
from jax.experimental import pallas as pl
from jax.experimental.pallas import tpu as pltpu

_F32 = jnp.float32


def _tabgnn_kernel(a_ref, x_ref, w0_ref, b0_ref, w1_ref, w0g_ref, w0x_ref,
                   bt_ref, mw0_ref, mb0_ref, wl_ref, bl_ref, o_ref,
                   scr_a, z_ref, xw_ref, c_ref, acc_ref, *, r, num_tiles,
                   num_graph_types):
    p = pl.program_id(0)
    t = pl.program_id(1)
    rows = pl.ds(t * r, r)

    @pl.when(p < num_graph_types)
    def _layer1():
        buf = p % 2

        @pl.when(t == 0)
        def _():
            xw_ref[...] = jnp.dot(x_ref[...], w0_ref[0],
                                  preferred_element_type=_F32)
            c_ref[...] = jnp.dot(w1_ref[0], w0g_ref[0],
                                 preferred_element_type=_F32)

        a_blk = a_ref[0]
        scr_a[buf, rows, :] = a_blk
        h1 = jnp.maximum(
            jnp.dot(a_blk, xw_ref[...], preferred_element_type=_F32)
            + b0_ref[0], 0.0)
        z_ref[buf, rows, :] = jnp.dot(h1, c_ref[...],
                                      preferred_element_type=_F32)

    @pl.when(p > 0)
    def _layer2():
        buf = (p - 1) % 2
        contrib = jnp.dot(scr_a[buf, rows, :], z_ref[buf],
                          preferred_element_type=_F32)

        @pl.when(p == 1)
        def _():
            acc_ref[rows, :] = (
                jnp.dot(x_ref[rows, :], w0x_ref[...],
                        preferred_element_type=_F32)
                + bt_ref[...] + contrib)

        @pl.when(p > 1)
        def _():
            acc_ref[rows, :] += contrib

        @pl.when(p == num_graph_types)
        def _():
            hm = jnp.maximum(acc_ref[rows, :], 0.0)
            hm = jnp.maximum(
                jnp.dot(hm, mw0_ref[...], preferred_element_type=_F32)
                + mb0_ref[...], 0.0)
            o_ref[...] = (jnp.sum(hm * wl_ref[...], axis=1, keepdims=True)
                          + bl_ref[...])


def kernel(a_hats, x, gnn_w_0, gnn_w_1, gnn_b_0, gnn_b_1, w0x, w0g, b0,
           mlp_w_0, mlp_b_0, mlp_w_1, mlp_b_1):
    G, N, _ = a_hats.shape
    Fin = x.shape[1]
    H = gnn_w_0.shape[2]
    Fout = gnn_w_1.shape[2]
    H0 = w0x.shape[1]
    Hm = mlp_w_0.shape[1]

    num_tiles = 4 if N % 4 == 0 else 1
    r = N // num_tiles

    bias_tot = b0
    for g in range(G):
        bias_tot = bias_tot + jnp.dot(gnn_b_1[g], w0g[g],
                                      preferred_element_type=_F32)

    gm1 = G - 1

    flops = int(G * (2 * N * Fin * H + 2 * N * N * H + 2 * N * H * H0
                     + 2 * N * N * H0)
                + 2 * N * Fin * H0 + 2 * N * H0 * Hm + 2 * N * Hm)
    bytes_accessed = int(4 * (G * N * N + N * Fin + N + Fin * H0
                              + G * (Fin * H + H + H * Fout + Fout * H0)))

    out = pl.pallas_call(
        functools.partial(_tabgnn_kernel, r=r, num_tiles=num_tiles,
                          num_graph_types=G),
        out_shape=jax.ShapeDtypeStruct((N, 1), _F32),
        grid=(G + 1, num_tiles),
        in_specs=[
            pl.BlockSpec((1, r, N),
                         lambda p, t: (jnp.minimum(p, gm1),
                                       jnp.where(p > gm1, num_tiles - 1, t),
                                       0)),
            pl.BlockSpec((N, Fin), lambda p, t: (0, 0)),
            pl.BlockSpec((1, Fin, H),
                         lambda p, t: (jnp.minimum(p, gm1), 0, 0)),
            pl.BlockSpec((1, 1, H),
                         lambda p, t: (jnp.minimum(p, gm1), 0, 0)),
            pl.BlockSpec((1, H, Fout),
                         lambda p, t: (jnp.minimum(p, gm1), 0, 0)),
            pl.BlockSpec((1, Fout, H0),
                         lambda p, t: (jnp.minimum(p, gm1), 0, 0)),
            pl.BlockSpec((Fin, H0), lambda p, t: (0, 0)),
            pl.BlockSpec((1, H0), lambda p, t: (0, 0)),
            pl.BlockSpec((H0, Hm), lambda p, t: (0, 0)),
            pl.BlockSpec((1, Hm), lambda p, t: (0, 0)),
            pl.BlockSpec((1, Hm), lambda p, t: (0, 0)),
            pl.BlockSpec((1, 1), lambda p, t: (0, 0)),
        ],
        out_specs=pl.BlockSpec(
            (r, 1), lambda p, t: (jnp.where(p > gm1, t, 0), 0)),
        scratch_shapes=[pltpu.VMEM((2, N, N), _F32),
                        pltpu.VMEM((2, N, H0), _F32),
                        pltpu.VMEM((N, H), _F32),
                        pltpu.VMEM((H, H0), _F32),
                        pltpu.VMEM((N, H0), _F32)],
        compiler_params=pltpu.CompilerParams(
            dimension_semantics=("arbitrary", "arbitrary"),
            vmem_limit_bytes=58 * 2**20),
        cost_estimate=pl.CostEstimate(flops=flops, transcendentals=0,
                                      bytes_accessed=bytes_accessed),
    )(a_hats, x, gnn_w_0, gnn_b_0, gnn_w_1, w0g, w0x, bias_tot,
      mlp_w_0, mlp_b_0, mlp_w_1, mlp_b_1)
    return out


# speedup vs baseline: 1.5347x; 1.0011x over previous
import functools

import jax
import jax.numpy as jnp
from jax.experimental import pallas as pl
from jax.experimental.pallas import tpu as pltpu

_F32 = jnp.float32


def _tabgnn_kernel(a_ref, x_ref, w0_ref, b0_ref, w1_ref, w0g_ref, w0x_ref,
                   bt_ref, mw0_ref, mb0_ref, wl_ref, bl_ref, o_ref,
                   scr_a, z_ref, xw_ref, c_ref, acc_ref, *, r, num_tiles,
                   num_graph_types):
    p = pl.program_id(0)
    t = pl.program_id(1)
    rows = pl.ds(t * r, r)

    @pl.when(p < num_graph_types)
    def _layer1():
        buf = p % 2

        @pl.when(t == 0)
        def _():
            xw_ref[...] = jnp.dot(x_ref[...], w0_ref[0],
                                  preferred_element_type=_F32
                                  ).astype(jnp.bfloat16)
            c_ref[...] = jnp.dot(w1_ref[0], w0g_ref[0],
                                 preferred_element_type=_F32
                                 ).astype(jnp.bfloat16)

        a16 = a_ref[0].astype(jnp.bfloat16)
        scr_a[buf, rows, :] = a16
        h1 = jnp.maximum(
            jnp.dot(a16, xw_ref[...], preferred_element_type=_F32)
            + b0_ref[0], 0.0)
        z_ref[buf, rows, :] = jnp.dot(
            h1.astype(jnp.bfloat16), c_ref[...],
            preferred_element_type=_F32).astype(jnp.bfloat16)

    @pl.when(p > 0)
    def _layer2():
        buf = (p - 1) % 2
        contrib = jnp.dot(scr_a[buf, rows, :], z_ref[buf],
                          preferred_element_type=_F32)

        @pl.when(p == 1)
        def _():
            acc_ref[rows, :] = (
                jnp.dot(x_ref[rows, :], w0x_ref[...],
                        preferred_element_type=_F32)
                + bt_ref[...] + contrib)

        @pl.when(p > 1)
        def _():
            acc_ref[rows, :] += contrib

        @pl.when(p == num_graph_types)
        def _():
            hm = jnp.maximum(acc_ref[rows, :], 0.0)
            hm = jnp.maximum(
                jnp.dot(hm, mw0_ref[...], preferred_element_type=_F32)
                + mb0_ref[...], 0.0)
            o_ref[...] = (jnp.sum(hm * wl_ref[...], axis=1, keepdims=True)
                          + bl_ref[...])


def kernel(a_hats, x, gnn_w_0, gnn_w_1, gnn_b_0, gnn_b_1, w0x, w0g, b0,
           mlp_w_0, mlp_b_0, mlp_w_1, mlp_b_1):
    G, N, _ = a_hats.shape
    Fin = x.shape[1]
    H = gnn_w_0.shape[2]
    Fout = gnn_w_1.shape[2]
    H0 = w0x.shape[1]
    Hm = mlp_w_0.shape[1]

    num_tiles = 4 if N % 4 == 0 else 1
    r = N // num_tiles

    bias_tot = b0
    for g in range(G):
        bias_tot = bias_tot + jnp.dot(gnn_b_1[g], w0g[g],
                                      preferred_element_type=_F32)

    gm1 = G - 1

    flops = int(G * (2 * N * Fin * H + 2 * N * N * H + 2 * N * H * H0
                     + 2 * N * N * H0)
                + 2 * N * Fin * H0 + 2 * N * H0 * Hm + 2 * N * Hm)
    bytes_accessed = int(4 * (G * N * N + N * Fin + N + Fin * H0
                              + G * (Fin * H + H + H * Fout + Fout * H0)))

    out = pl.pallas_call(
        functools.partial(_tabgnn_kernel, r=r, num_tiles=num_tiles,
                          num_graph_types=G),
        out_shape=jax.ShapeDtypeStruct((N, 1), _F32),
        grid=(G + 1, num_tiles),
        in_specs=[
            pl.BlockSpec((1, r, N),
                         lambda p, t: (jnp.minimum(p, gm1),
                                       jnp.where(p > gm1, num_tiles - 1, t),
                                       0)),
            pl.BlockSpec((N, Fin), lambda p, t: (0, 0)),
            pl.BlockSpec((1, Fin, H),
                         lambda p, t: (jnp.minimum(p, gm1), 0, 0)),
            pl.BlockSpec((1, 1, H),
                         lambda p, t: (jnp.minimum(p, gm1), 0, 0)),
            pl.BlockSpec((1, H, Fout),
                         lambda p, t: (jnp.minimum(p, gm1), 0, 0)),
            pl.BlockSpec((1, Fout, H0),
                         lambda p, t: (jnp.minimum(p, gm1), 0, 0)),
            pl.BlockSpec((Fin, H0), lambda p, t: (0, 0)),
            pl.BlockSpec((1, H0), lambda p, t: (0, 0)),
            pl.BlockSpec((H0, Hm), lambda p, t: (0, 0)),
            pl.BlockSpec((1, Hm), lambda p, t: (0, 0)),
            pl.BlockSpec((1, Hm), lambda p, t: (0, 0)),
            pl.BlockSpec((1, 1), lambda p, t: (0, 0)),
        ],
        out_specs=pl.BlockSpec(
            (r, 1), lambda p, t: (jnp.where(p > gm1, t, 0), 0)),
        scratch_shapes=[pltpu.VMEM((2, N, N), jnp.bfloat16),
                        pltpu.VMEM((2, N, H0), jnp.bfloat16),
                        pltpu.VMEM((N, H), jnp.bfloat16),
                        pltpu.VMEM((H, H0), jnp.bfloat16),
                        pltpu.VMEM((N, H0), _F32)],
        compiler_params=pltpu.CompilerParams(
            dimension_semantics=("arbitrary", "arbitrary"),
            vmem_limit_bytes=58 * 2**20),
        cost_estimate=pl.CostEstimate(flops=flops, transcendentals=0,
                                      bytes_accessed=bytes_accessed),
    )(a_hats, x, gnn_w_0, gnn_b_0, gnn_w_1, w0g, w0x, bias_tot,
      mlp_w_0, mlp_b_0, mlp_w_1, mlp_b_1)
    return out
